# baseline (speedup 1.0000x reference)
import functools
import math

import numpy as np
import jax
import jax.numpy as jnp
from jax import lax
from jax.experimental import pallas as pl
from jax.experimental.pallas import tpu as pltpu
from jax.experimental.pallas import tpu_sc as plsc

SIZE = 1024
CELL = 8
J = SIZE // CELL
DMODEL = 768
NLEV = 8
VOCAB = 1024
THR = 0.95
B = 128

TB = 32
TB3 = 8


def _build_haar_T() -> np.ndarray:
    T = np.zeros((SIZE, NLEV * J), dtype=np.float32)
    for k in range(NLEV):
        D = 10 - k
        blk = 1 << D
        scale = np.float32(2.0 ** (-D / 2.0))
        for j in range(J):
            m = j >> (7 - k)
            base = m * blk
            T[base:base + blk // 2, k * J + j] = scale
            T[base + blk // 2:base + blk, k * J + j] = -scale
    return T


def _build_pos_emb() -> np.ndarray:
    position = np.arange(J, dtype=np.float32)[:, None]
    div = np.exp(np.arange(0, DMODEL, 2, dtype=np.float32)
                 * (-math.log(10000.0) / DMODEL))
    pe = np.zeros((J, DMODEL), dtype=np.float32)
    pe[:, 0::2] = np.sin(position * div)
    pe[:, 1::2] = np.cos(position * div)
    return pe


_HAAR_T = _build_haar_T()
_POS_EMB = _build_pos_emb()


def _shift_up(a, k, fill):
    pad = jnp.full((k,) + a.shape[1:], fill, a.dtype)
    return jnp.concatenate([a[k:], pad], axis=0)


def _corr_kernel(x_ref, xs_ref, w_ref, t_ref, bi_ref, mask_ref, hc_ref):
    words = w_ref[...]

    mw = jnp.mean(words, axis=1, keepdims=True)
    wc = words - mw
    sw = jnp.sqrt(jnp.mean(wc * wc, axis=1, keepdims=True))

    seg = xs_ref[...]
    ms = jnp.mean(seg, axis=1, keepdims=True)
    sc = seg - ms

    ones_row = jnp.ones((1, CELL), jnp.float32)
    ss2 = lax.dot_general(ones_row, sc * sc, (((1,), (1,)), ((), ())),
                          preferred_element_type=jnp.float32,
                          precision=lax.Precision.HIGHEST)
    ssr = jnp.sqrt(ss2 / CELL)

    cov = lax.dot_general(wc, sc, (((1,), (1,)), ((), ())),
                          preferred_element_type=jnp.float32) / CELL
    denom = sw * ssr
    safe = denom != 0.0
    r = jnp.where(safe, cov / jnp.where(safe, denom, 1.0), 0.0)

    av = jnp.abs(r)
    A = jnp.where(av > THR, av, -1.0)

    S = _shift_up(A, 1, -1.0)
    k = 1
    while k < VOCAB:
        S = jnp.maximum(S, _shift_up(S, k, -1.0))
        k *= 2

    event = A > 0.0
    posr = r > 0.0
    cand = event & ((posr & (A >= S)) | ((~posr) & (S <= THR)))

    iota_v = lax.broadcasted_iota(jnp.int32, r.shape, 0)
    pick = jnp.where(cand, iota_v, 2047)
    first = jnp.min(pick, axis=0)
    found = first < VOCAB
    spread = lax.broadcasted_iota(jnp.int32, (TB * J,), 0)
    bi_ref[...] = jnp.where(found, first, spread)
    mask_ref[...] = jnp.where(found, 1.0, 0.0)

    hc_ref[...] = jnp.dot(x_ref[...], t_ref[...],
                          preferred_element_type=jnp.float32,
                          precision=lax.Precision.HIGHEST)


def _run_corr(X, Xs, words):
    grid = B // TB
    return pl.pallas_call(
        _corr_kernel,
        grid=(grid,),
        in_specs=[
            pl.BlockSpec((TB, SIZE), lambda i: (i, 0)),
            pl.BlockSpec((TB * J, CELL), lambda i: (i, 0)),
            pl.BlockSpec((VOCAB, CELL), lambda i: (0, 0)),
            pl.BlockSpec((SIZE, NLEV * J), lambda i: (0, 0)),
        ],
        out_specs=[
            pl.BlockSpec((TB * J,), lambda i: (i,)),
            pl.BlockSpec((TB * J,), lambda i: (i,)),
            pl.BlockSpec((TB, NLEV * J), lambda i: (i, 0)),
        ],
        out_shape=[
            jax.ShapeDtypeStruct((B * J,), jnp.int32),
            jax.ShapeDtypeStruct((B * J,), jnp.float32),
            jax.ShapeDtypeStruct((B, NLEV * J), jnp.float32),
        ],
    )(X, Xs, words, jnp.asarray(_HAAR_T))


_NC = 2
_NS = 16
_NW = _NC * _NS
_ROWS = B * J
_RPW = _ROWS // _NW
_CHUNK = 64
_NCHUNK = _RPW // _CHUNK


def _sc_gather(table, idx):
    idx3 = idx.reshape(_NW, _NCHUNK, _CHUNK)
    mesh = plsc.VectorSubcoreMesh(core_axis_name="c", subcore_axis_name="s")

    @functools.partial(
        pl.kernel,
        mesh=mesh,
        out_type=jax.ShapeDtypeStruct((_ROWS, DMODEL), jnp.float32),
        scratch_types=[
            pltpu.VMEM((_NCHUNK, _CHUNK), jnp.int32),
            pltpu.VMEM((_CHUNK, DMODEL), jnp.float32),
            pltpu.VMEM((_CHUNK, DMODEL), jnp.float32),
            pltpu.SemaphoreType.DMA,
            pltpu.SemaphoreType.DMA,
            pltpu.SemaphoreType.DMA,
            pltpu.SemaphoreType.DMA,
        ],
    )
    def gather_kernel(table_hbm, idx_hbm, out_hbm, idx_v, rows0, rows1,
                      gsem0, gsem1, osem0, osem1):
        wid = lax.axis_index("s") * _NC + lax.axis_index("c")
        pltpu.sync_copy(idx_hbm.at[wid], idx_v)
        base = wid * _RPW
        bufs = (rows0, rows1)
        gsems = (gsem0, gsem1)
        osems = (osem0, osem1)
        gathers = [None] * _NCHUNK
        outs = [None] * _NCHUNK
        gathers[0] = pltpu.async_copy(
            table_hbm.at[idx_v.at[0]], bufs[0], gsems[0])
        for c in range(_NCHUNK):
            n = c + 1
            if n < _NCHUNK:
                if c >= 1:
                    outs[c - 1].wait()
                gathers[n] = pltpu.async_copy(
                    table_hbm.at[idx_v.at[n]], bufs[n % 2], gsems[n % 2])
            gathers[c].wait()
            outs[c] = pltpu.async_copy(
                bufs[c % 2], out_hbm.at[pl.ds(base + c * _CHUNK, _CHUNK)],
                osems[c % 2])
        outs[_NCHUNK - 2].wait()
        outs[_NCHUNK - 1].wait()

    return gather_kernel(table, idx3)


def _combine_kernel(emb_ref, hck_ref, mask_ref, wh_ref, pos_ref, out_ref):
    n = TB3 * J
    row_j = lax.broadcasted_iota(jnp.int32, (n, J), 0) % J
    lane_j = lax.broadcasted_iota(jnp.int32, (n, J), 1)
    dsel = row_j == lane_j

    hck = hck_ref[...]
    cols = []
    for k in range(NLEV):
        fk = hck[:, k * J:(k + 1) * J]
        yk = jnp.broadcast_to(fk[:, None, :], (TB3, J, J)).reshape(n, J)
        cols.append(jnp.sum(jnp.where(dsel, yk, 0.0), axis=1, keepdims=True))
    hc2 = jnp.concatenate(cols, axis=1)

    mask_blk = mask_ref[...]
    ym = jnp.broadcast_to(mask_blk[:, None, :], (TB3, J, J)).reshape(n, J)
    mcol = jnp.sum(jnp.where(dsel, ym, 0.0), axis=1, keepdims=True)

    he = jnp.dot(hc2, wh_ref[...], preferred_element_type=jnp.float32)
    emb = emb_ref[...].reshape(n, DMODEL)
    pos = jnp.broadcast_to(pos_ref[...][None], (TB3, J, DMODEL)).reshape(n, DMODEL)
    out = (emb + he) * mcol + pos
    out_ref[...] = out.reshape(TB3, J, DMODEL)


def _run_combine(emb, hcK, mask2d, W_haar):
    grid = B // TB3
    return pl.pallas_call(
        _combine_kernel,
        grid=(grid,),
        in_specs=[
            pl.BlockSpec((TB3, J, DMODEL), lambda i: (i, 0, 0)),
            pl.BlockSpec((TB3, NLEV * J), lambda i: (i, 0)),
            pl.BlockSpec((TB3, J), lambda i: (i, 0)),
            pl.BlockSpec((NLEV, DMODEL), lambda i: (0, 0)),
            pl.BlockSpec((J, DMODEL), lambda i: (0, 0)),
        ],
        out_specs=pl.BlockSpec((TB3, J, DMODEL), lambda i: (i, 0, 0)),
        out_shape=jax.ShapeDtypeStruct((B, J, DMODEL), jnp.float32),
    )(emb, hcK, mask2d, W_haar, jnp.asarray(_POS_EMB))


def kernel(X, words, W_word, W_haar):
    Xs = X.reshape(B * J, CELL)
    bi, mask, hcK = _run_corr(X, Xs, words)
    emb = _sc_gather(W_word, bi)
    emb = emb.reshape(B, J, DMODEL)
    return _run_combine(emb, hcK, mask.reshape(B, J), W_haar)


# speedup vs baseline: 1.1932x; 1.1932x over previous
import functools
import math

import numpy as np
import jax
import jax.numpy as jnp
from jax import lax
from jax.experimental import pallas as pl
from jax.experimental.pallas import tpu as pltpu
from jax.experimental.pallas import tpu_sc as plsc

SIZE = 1024
CELL = 8
J = SIZE // CELL
DMODEL = 768
NLEV = 8
VOCAB = 1024
THR = 0.95
B = 128

TB = 16
TB3 = 16


def _build_haar_T() -> np.ndarray:
    T = np.zeros((SIZE, NLEV * J), dtype=np.float32)
    for k in range(NLEV):
        D = 10 - k
        blk = 1 << D
        scale = np.float32(2.0 ** (-D / 2.0))
        for j in range(J):
            m = j >> (7 - k)
            base = m * blk
            T[base:base + blk // 2, k * J + j] = scale
            T[base + blk // 2:base + blk, k * J + j] = -scale
    return T


def _build_pos_emb() -> np.ndarray:
    position = np.arange(J, dtype=np.float32)[:, None]
    div = np.exp(np.arange(0, DMODEL, 2, dtype=np.float32)
                 * (-math.log(10000.0) / DMODEL))
    pe = np.zeros((J, DMODEL), dtype=np.float32)
    pe[:, 0::2] = np.sin(position * div)
    pe[:, 1::2] = np.cos(position * div)
    return pe


_HAAR_T = _build_haar_T()
_POS_EMB = _build_pos_emb()


def _shift_up(a, k, fill):
    pad = jnp.full((k,) + a.shape[1:], fill, a.dtype)
    return jnp.concatenate([a[k:], pad], axis=0)


def _corr_kernel(x_ref, xs_ref, w_ref, t_ref, bi_ref, mask_ref, hc_ref):
    words = w_ref[...]

    mw = jnp.mean(words, axis=1, keepdims=True)
    wc = words - mw
    sw = jnp.sqrt(jnp.mean(wc * wc, axis=1, keepdims=True))

    seg = xs_ref[...]
    ms = jnp.mean(seg, axis=1, keepdims=True)
    sc = seg - ms

    ones_row = jnp.ones((1, CELL), jnp.float32)
    ss2 = lax.dot_general(ones_row, sc * sc, (((1,), (1,)), ((), ())),
                          preferred_element_type=jnp.float32,
                          precision=lax.Precision.HIGHEST)
    ssr = jnp.sqrt(ss2 / CELL)

    cov = lax.dot_general(wc, sc, (((1,), (1,)), ((), ())),
                          preferred_element_type=jnp.float32) / CELL
    denom = sw * ssr
    safe = denom != 0.0
    r = jnp.where(safe, cov / jnp.where(safe, denom, 1.0), 0.0)

    av = jnp.abs(r)
    A = jnp.where(av > THR, av, -1.0)

    S = _shift_up(A, 1, -1.0)
    k = 1
    while k < VOCAB:
        S = jnp.maximum(S, _shift_up(S, k, -1.0))
        k *= 2

    event = A > 0.0
    posr = r > 0.0
    cand = event & ((posr & (A >= S)) | ((~posr) & (S <= THR)))

    iota_v = lax.broadcasted_iota(jnp.int32, r.shape, 0)
    pick = jnp.where(cand, iota_v, 2047)
    first = jnp.min(pick, axis=0)
    found = first < VOCAB
    spread = lax.broadcasted_iota(jnp.int32, (TB * J,), 0)
    bi_ref[...] = jnp.where(found, first, spread)
    mask_ref[...] = jnp.where(found, 1.0, 0.0)

    hc_ref[...] = jnp.dot(x_ref[...], t_ref[...],
                          preferred_element_type=jnp.float32,
                          precision=lax.Precision.HIGHEST)


def _run_corr(X, Xs, words):
    grid = B // TB
    return pl.pallas_call(
        _corr_kernel,
        grid=(grid,),
        in_specs=[
            pl.BlockSpec((TB, SIZE), lambda i: (i, 0)),
            pl.BlockSpec((TB * J, CELL), lambda i: (i, 0)),
            pl.BlockSpec((VOCAB, CELL), lambda i: (0, 0)),
            pl.BlockSpec((SIZE, NLEV * J), lambda i: (0, 0)),
        ],
        out_specs=[
            pl.BlockSpec((TB * J,), lambda i: (i,)),
            pl.BlockSpec((TB * J,), lambda i: (i,)),
            pl.BlockSpec((TB, NLEV * J), lambda i: (i, 0)),
        ],
        out_shape=[
            jax.ShapeDtypeStruct((B * J,), jnp.int32),
            jax.ShapeDtypeStruct((B * J,), jnp.float32),
            jax.ShapeDtypeStruct((B, NLEV * J), jnp.float32),
        ],
    )(X, Xs, words, jnp.asarray(_HAAR_T))


_NC = 2
_NS = 16
_NW = _NC * _NS
_ROWS = B * J
_RPW = _ROWS // _NW
_CHUNK = 64
_NCHUNK = _RPW // _CHUNK


def _sc_gather(table, idx):
    idx3 = idx.reshape(_NW, _NCHUNK, _CHUNK)
    mesh = plsc.VectorSubcoreMesh(core_axis_name="c", subcore_axis_name="s")

    @functools.partial(
        pl.kernel,
        mesh=mesh,
        out_type=jax.ShapeDtypeStruct((_ROWS, DMODEL), jnp.float32),
        scratch_types=[
            pltpu.VMEM((_NCHUNK, _CHUNK), jnp.int32),
            pltpu.VMEM((_CHUNK, DMODEL), jnp.float32),
            pltpu.VMEM((_CHUNK, DMODEL), jnp.float32),
            pltpu.SemaphoreType.DMA,
            pltpu.SemaphoreType.DMA,
            pltpu.SemaphoreType.DMA,
            pltpu.SemaphoreType.DMA,
        ],
    )
    def gather_kernel(table_hbm, idx_hbm, out_hbm, idx_v, rows0, rows1,
                      gsem0, gsem1, osem0, osem1):
        wid = lax.axis_index("s") * _NC + lax.axis_index("c")
        pltpu.sync_copy(idx_hbm.at[wid], idx_v)
        base = wid * _RPW
        bufs = (rows0, rows1)
        gsems = (gsem0, gsem1)
        osems = (osem0, osem1)
        gathers = [None] * _NCHUNK
        outs = [None] * _NCHUNK
        gathers[0] = pltpu.async_copy(
            table_hbm.at[idx_v.at[0]], bufs[0], gsems[0])
        for c in range(_NCHUNK):
            n = c + 1
            if n < _NCHUNK:
                if c >= 1:
                    outs[c - 1].wait()
                gathers[n] = pltpu.async_copy(
                    table_hbm.at[idx_v.at[n]], bufs[n % 2], gsems[n % 2])
            gathers[c].wait()
            outs[c] = pltpu.async_copy(
                bufs[c % 2], out_hbm.at[pl.ds(base + c * _CHUNK, _CHUNK)],
                osems[c % 2])
        outs[_NCHUNK - 2].wait()
        outs[_NCHUNK - 1].wait()

    return gather_kernel(table, idx3)


def _combine_kernel(emb_ref, hck_ref, mask_ref, wh_ref, pos_ref, out_ref):
    n = TB3 * J
    row_j = lax.broadcasted_iota(jnp.int32, (n, J), 0) % J
    lane_j = lax.broadcasted_iota(jnp.int32, (n, J), 1)
    dsel = row_j == lane_j

    hck = hck_ref[...]
    cols = []
    for k in range(NLEV):
        fk = hck[:, k * J:(k + 1) * J]
        yk = jnp.broadcast_to(fk[:, None, :], (TB3, J, J)).reshape(n, J)
        cols.append(jnp.sum(jnp.where(dsel, yk, 0.0), axis=1, keepdims=True))
    hc2 = jnp.concatenate(cols, axis=1)

    mask_blk = mask_ref[...]
    ym = jnp.broadcast_to(mask_blk[:, None, :], (TB3, J, J)).reshape(n, J)
    mcol = jnp.sum(jnp.where(dsel, ym, 0.0), axis=1, keepdims=True)

    he = jnp.dot(hc2, wh_ref[...], preferred_element_type=jnp.float32)
    emb = emb_ref[...].reshape(n, DMODEL)
    pos = jnp.broadcast_to(pos_ref[...][None], (TB3, J, DMODEL)).reshape(n, DMODEL)
    out = (emb + he) * mcol + pos
    out_ref[...] = out.reshape(TB3, J, DMODEL)


def _run_combine(emb, hcK, mask2d, W_haar):
    grid = B // TB3
    return pl.pallas_call(
        _combine_kernel,
        grid=(grid,),
        in_specs=[
            pl.BlockSpec((TB3, J, DMODEL), lambda i: (i, 0, 0)),
            pl.BlockSpec((TB3, NLEV * J), lambda i: (i, 0)),
            pl.BlockSpec((TB3, J), lambda i: (i, 0)),
            pl.BlockSpec((NLEV, DMODEL), lambda i: (0, 0)),
            pl.BlockSpec((J, DMODEL), lambda i: (0, 0)),
        ],
        out_specs=pl.BlockSpec((TB3, J, DMODEL), lambda i: (i, 0, 0)),
        out_shape=jax.ShapeDtypeStruct((B, J, DMODEL), jnp.float32),
    )(emb, hcK, mask2d, W_haar, jnp.asarray(_POS_EMB))


def kernel(X, words, W_word, W_haar):
    Xs = X.reshape(B * J, CELL)
    bi, mask, hcK = _run_corr(X, Xs, words)
    emb = _sc_gather(W_word, bi)
    emb = emb.reshape(B, J, DMODEL)
    return _run_combine(emb, hcK, mask.reshape(B, J), W_haar)
